# baseline (speedup 1.0000x reference)
import functools

import jax
import jax.numpy as jnp
from jax import lax
from jax.experimental import pallas as pl
from jax.experimental.pallas import tpu as pltpu
from jax.experimental.pallas import tpu_sc as plsc

_TEST = 0.0
_NC, _NS, _L = 2, 16, 16
_NW = _NC * _NS
_CHUNK = 64
_NBUF = 2


def _fix_col0_group(c_v, d_v, g, iota, zeros):
    rows = g * _L + iota
    c0 = plsc.load_gather(c_v, [rows, zeros])
    d0 = plsc.load_gather(d_v, [rows, zeros])
    lo = c0 - d0
    hi = c0 + d0
    left = lo <= _TEST
    right = hi > _TEST
    min_hi = jnp.minimum(hi, _TEST)
    max_lo = jnp.maximum(lo, _TEST)
    cl = (lo + min_hi) * 0.5
    dl = (min_hi - lo) * 0.5
    cr = (max_lo + hi) * 0.5
    dr = (hi - max_lo) * 0.5
    l_join = jnp.minimum(cl - dl, cr - dr)
    r_join = jnp.maximum(cl + dl, cr + dr)
    cb = (l_join + r_join) * 0.5
    db = (r_join - l_join) * 0.5
    both = left & right
    new_c0 = jnp.where(both, cb, jnp.where(left, cl, cr))
    new_d0 = jnp.where(both, db, jnp.where(left, dl, dr))
    plsc.store_scatter(c_v, [rows, zeros], new_c0)
    plsc.store_scatter(d_v, [rows, zeros], new_d0)


def _sc_call(c, d):
    n, f = c.shape
    rows_per_w = n // _NW
    n_chunks = rows_per_w // _CHUNK
    n_groups = n_chunks // _NBUF
    mesh = plsc.VectorSubcoreMesh(core_axis_name="c", subcore_axis_name="s")

    @functools.partial(
        pl.kernel,
        out_type=[
            jax.ShapeDtypeStruct((n, f), jnp.float32),
            jax.ShapeDtypeStruct((n, f), jnp.float32),
        ],
        mesh=mesh,
        scratch_types=(
            [pltpu.VMEM((_CHUNK, f), jnp.float32) for _ in range(2 * _NBUF)]
            + [pltpu.VMEM((_CHUNK,), jnp.int32) for _ in range(2 * _NBUF)]
            + [pltpu.SemaphoreType.DMA for _ in range(2 * _NBUF)]
        ),
        compiler_params=pltpu.CompilerParams(needs_layout_passes=False),
    )
    def run(c_hbm, d_hbm, oc_hbm, od_hbm, *scr):
        c_bufs = scr[0:_NBUF]
        d_bufs = scr[_NBUF:2 * _NBUF]
        in_idx = scr[2 * _NBUF:3 * _NBUF]
        out_idx = scr[3 * _NBUF:4 * _NBUF]
        in_sems = scr[4 * _NBUF:5 * _NBUF]
        out_sems = scr[5 * _NBUF:6 * _NBUF]
        wid = lax.axis_index("s") * _NC + lax.axis_index("c")
        base0 = wid * rows_per_w
        iota = lax.iota(jnp.int32, _L)
        zeros = jnp.zeros((_L,), jnp.int32)

        def set_idx(idx_ref, j):
            base = base0 + j * _CHUNK
            for u in range(_CHUNK // _L):
                idx_ref[pl.ds(u * _L, _L)] = base + u * _L + iota

        def fire_in(j, b):
            set_idx(in_idx[b], j)
            pltpu.async_copy(c_hbm.at[in_idx[b]], c_bufs[b], in_sems[b])
            pltpu.async_copy(d_hbm.at[in_idx[b]], d_bufs[b], in_sems[b])

        def wait_in(b):
            pltpu.make_async_copy(c_hbm.at[in_idx[b]], c_bufs[b], in_sems[b]).wait()
            pltpu.make_async_copy(d_hbm.at[in_idx[b]], d_bufs[b], in_sems[b]).wait()

        def fire_out(j, b):
            set_idx(out_idx[b], j)
            pltpu.async_copy(c_bufs[b], oc_hbm.at[out_idx[b]], out_sems[b])
            pltpu.async_copy(d_bufs[b], od_hbm.at[out_idx[b]], out_sems[b])

        def wait_out(b):
            pltpu.make_async_copy(c_bufs[b], oc_hbm.at[out_idx[b]], out_sems[b]).wait()
            pltpu.make_async_copy(d_bufs[b], od_hbm.at[out_idx[b]], out_sems[b]).wait()

        def process(j, b):
            wait_in(b)
            for g in range(_CHUNK // _L):
                _fix_col0_group(c_bufs[b], d_bufs[b], g, iota, zeros)
            fire_out(j, b)

        for b in range(_NBUF):
            fire_in(b, b)

        def ring_cycle(kg, carry):
            for b in range(_NBUF):
                j = kg * _NBUF + b
                process(j, b)
                wait_out(b)
                fire_in(j + _NBUF, b)
            return carry

        lax.fori_loop(0, n_groups - 1, ring_cycle, 0)

        for b in range(_NBUF):
            j = (n_groups - 1) * _NBUF + b
            process(j, b)
            wait_out(b)

    return run(c, d)


def kernel(c, delta, idx):
    out_c, out_d = _sc_call(c, delta)
    return out_c, out_d


# speedup vs baseline: 1.0171x; 1.0171x over previous
import functools

import jax
import jax.numpy as jnp
from jax import lax
from jax.experimental import pallas as pl
from jax.experimental.pallas import tpu as pltpu
from jax.experimental.pallas import tpu_sc as plsc

_TEST = 0.0
_NC, _NS, _L = 2, 16, 16
_NW = _NC * _NS
_CHUNK = 64
_NBUF = 2


def _fix_col0_group(c_v, d_v, g, iota, zeros):
    rows = g * _L + iota
    c0 = plsc.load_gather(c_v, [rows, zeros])
    d0 = plsc.load_gather(d_v, [rows, zeros])
    lo = c0 - d0
    hi = c0 + d0
    left = lo <= _TEST
    right = hi > _TEST
    min_hi = jnp.minimum(hi, _TEST)
    max_lo = jnp.maximum(lo, _TEST)
    cl = (lo + min_hi) * 0.5
    dl = (min_hi - lo) * 0.5
    cr = (max_lo + hi) * 0.5
    dr = (hi - max_lo) * 0.5
    l_join = jnp.minimum(cl - dl, cr - dr)
    r_join = jnp.maximum(cl + dl, cr + dr)
    cb = (l_join + r_join) * 0.5
    db = (r_join - l_join) * 0.5
    both = left & right
    new_c0 = jnp.where(both, cb, jnp.where(left, cl, cr))
    new_d0 = jnp.where(both, db, jnp.where(left, dl, dr))
    plsc.store_scatter(c_v, [rows, zeros], new_c0)
    plsc.store_scatter(d_v, [rows, zeros], new_d0)


def _sc_call(c, d):
    n, f = c.shape
    rows_per_w = n // _NW
    n_chunks = rows_per_w // _CHUNK
    n_groups = n_chunks // _NBUF
    mesh = plsc.VectorSubcoreMesh(core_axis_name="c", subcore_axis_name="s")

    @functools.partial(
        pl.kernel,
        out_type=[
            jax.ShapeDtypeStruct((n, f), jnp.float32),
            jax.ShapeDtypeStruct((n, f), jnp.float32),
        ],
        mesh=mesh,
        scratch_types=(
            [pltpu.VMEM((_CHUNK, f), jnp.float32) for _ in range(2 * _NBUF)]
            + [pltpu.SemaphoreType.DMA for _ in range(2 * _NBUF)]
        ),
        compiler_params=pltpu.CompilerParams(needs_layout_passes=False),
    )
    def run(c_hbm, d_hbm, oc_hbm, od_hbm, *scr):
        c_bufs = scr[0:_NBUF]
        d_bufs = scr[_NBUF:2 * _NBUF]
        in_sems = scr[2 * _NBUF:3 * _NBUF]
        out_sems = scr[3 * _NBUF:4 * _NBUF]
        wid = lax.axis_index("s") * _NC + lax.axis_index("c")
        base0 = wid * rows_per_w
        iota = lax.iota(jnp.int32, _L)
        zeros = jnp.zeros((_L,), jnp.int32)

        def fire_in(j, b):
            base = base0 + j * _CHUNK
            pltpu.async_copy(c_hbm.at[pl.ds(base, _CHUNK)], c_bufs[b], in_sems[b])
            pltpu.async_copy(d_hbm.at[pl.ds(base, _CHUNK)], d_bufs[b], in_sems[b])

        def wait_in(j, b):
            base = base0 + j * _CHUNK
            pltpu.make_async_copy(c_hbm.at[pl.ds(base, _CHUNK)], c_bufs[b], in_sems[b]).wait()
            pltpu.make_async_copy(d_hbm.at[pl.ds(base, _CHUNK)], d_bufs[b], in_sems[b]).wait()

        def fire_out(j, b):
            base = base0 + j * _CHUNK
            pltpu.async_copy(c_bufs[b], oc_hbm.at[pl.ds(base, _CHUNK)], out_sems[b])
            pltpu.async_copy(d_bufs[b], od_hbm.at[pl.ds(base, _CHUNK)], out_sems[b])

        def wait_out(j, b):
            base = base0 + j * _CHUNK
            pltpu.make_async_copy(c_bufs[b], oc_hbm.at[pl.ds(base, _CHUNK)], out_sems[b]).wait()
            pltpu.make_async_copy(d_bufs[b], od_hbm.at[pl.ds(base, _CHUNK)], out_sems[b]).wait()

        def process(j, b):
            wait_in(j, b)
            for g in range(_CHUNK // _L):
                _fix_col0_group(c_bufs[b], d_bufs[b], g, iota, zeros)
            fire_out(j, b)

        for b in range(_NBUF):
            fire_in(b, b)

        def ring_cycle(kg, carry):
            for b in range(_NBUF):
                j = kg * _NBUF + b
                process(j, b)
                wait_out(j, b)
                fire_in(j + _NBUF, b)
            return carry

        lax.fori_loop(0, n_groups - 1, ring_cycle, 0)

        for b in range(_NBUF):
            j = (n_groups - 1) * _NBUF + b
            process(j, b)
            wait_out(j, b)

    return run(c, d)


def kernel(c, delta, idx):
    out_c, out_d = _sc_call(c, delta)
    return out_c, out_d
